# baseline (speedup 1.0000x reference)
import functools
import math

import jax
import jax.numpy as jnp
from jax import lax
from jax.experimental import pallas as pl
from jax.experimental.pallas import tpu as pltpu
from jax.experimental.pallas import tpu_sc as plsc

N_TOK = 8192
D_MODEL = 1024
D_FF = 4096
N_EXP = 8
TOPK = 2
CAP = 2048
DUMMY = N_EXP * CAP
XG_ROWS = N_EXP * CAP + CAP

TOK_BLK = 256
N_TOK_BLKS = N_TOK // TOK_BLK
FF_BLK = 1024
N_FF_BLKS = D_FF // FF_BLK
ROW_BLK = 256

SC_CHUNK = 32
_RSQRT2 = 1.0 / math.sqrt(2.0)


def _router_body(x_ref, wg_ref, s1_ref, s2_ref, w1_ref, w2_ref, carry_ref):
    i = pl.program_id(0)

    @pl.when(i == 0)
    def _():
        carry_ref[...] = jnp.zeros((1, N_EXP), jnp.float32)

    x = x_ref[...]
    wg = wg_ref[...]
    logits = lax.dot_general(
        x, wg, (((1,), (1,)), ((), ())), preferred_element_type=jnp.float32
    )
    z = logits - jnp.max(logits, axis=1, keepdims=True)
    p = jnp.exp(z)
    gates = p / jnp.sum(p, axis=1, keepdims=True)

    e_iota = lax.broadcasted_iota(jnp.int32, (TOK_BLK, N_EXP), 1)
    v1 = jnp.max(gates, axis=1, keepdims=True)
    i1 = jnp.min(jnp.where(gates == v1, e_iota, N_EXP), axis=1, keepdims=True)
    oh1 = e_iota == i1
    g2 = jnp.where(oh1, -1.0, gates)
    v2 = jnp.max(g2, axis=1, keepdims=True)
    i2 = jnp.min(jnp.where(g2 == v2, e_iota, N_EXP), axis=1, keepdims=True)
    oh2 = e_iota == i2

    mf = (oh1 | oh2).astype(jnp.float32)
    row = lax.broadcasted_iota(jnp.int32, (TOK_BLK, TOK_BLK), 0)
    col = lax.broadcasted_iota(jnp.int32, (TOK_BLK, TOK_BLK), 1)
    trif = (row >= col).astype(jnp.float32)
    incl = lax.dot_general(
        trif, mf, (((1,), (0,)), ((), ())), preferred_element_type=jnp.float32
    )
    pos = carry_ref[...] + incl - 1.0
    keepf = jnp.where(pos < float(CAP), mf, 0.0)
    gsf = e_iota.astype(jnp.float32) * float(CAP) + pos
    gs = jnp.where(keepf > 0, gsf, float(DUMMY))

    slot1 = jnp.sum(jnp.where(oh1, gs, 0.0), axis=1, keepdims=True)
    slot2 = jnp.sum(jnp.where(oh2, gs, 0.0), axis=1, keepdims=True)
    k1 = jnp.sum(jnp.where(oh1, keepf, 0.0), axis=1, keepdims=True)
    k2 = jnp.sum(jnp.where(oh2, keepf, 0.0), axis=1, keepdims=True)

    s1_ref[...] = slot1.astype(jnp.int32)
    s2_ref[...] = slot2.astype(jnp.int32)
    w1_ref[...] = jnp.broadcast_to(v1 * k1, (TOK_BLK, 16))
    w2_ref[...] = jnp.broadcast_to(v2 * k2, (TOK_BLK, 16))
    carry_ref[...] = carry_ref[...] + jnp.sum(mf, axis=0, keepdims=True)


def _router(flat, wg):
    return pl.pallas_call(
        _router_body,
        grid=(N_TOK_BLKS,),
        in_specs=[
            pl.BlockSpec((TOK_BLK, D_MODEL), lambda i: (i, 0)),
            pl.BlockSpec((N_EXP, D_MODEL), lambda i: (0, 0)),
        ],
        out_specs=[
            pl.BlockSpec((TOK_BLK, 1), lambda i: (i, 0)),
            pl.BlockSpec((TOK_BLK, 1), lambda i: (i, 0)),
            pl.BlockSpec((TOK_BLK, 16), lambda i: (i, 0)),
            pl.BlockSpec((TOK_BLK, 16), lambda i: (i, 0)),
        ],
        out_shape=[
            jax.ShapeDtypeStruct((N_TOK, 1), jnp.int32),
            jax.ShapeDtypeStruct((N_TOK, 1), jnp.int32),
            jax.ShapeDtypeStruct((N_TOK, 16), jnp.float32),
            jax.ShapeDtypeStruct((N_TOK, 16), jnp.float32),
        ],
        scratch_shapes=[pltpu.VMEM((1, N_EXP), jnp.float32)],
    )(flat, wg)


_N_DCH = (N_TOK // 32) // SC_CHUNK


def _dispatch_body(fbf_hbm, s1_hbm, s2_hbm, xg_hbm, idx1, idx2, rows, ldsem, scsem):
    wid = lax.axis_index("c") * 16 + lax.axis_index("s")
    base = wid * (N_TOK // 32)

    def tok0(k):
        return base + k * SC_CHUNK

    h_load = {}
    h_sc = {}
    h_load[0] = pltpu.async_copy(fbf_hbm.at[pl.ds(tok0(0), SC_CHUNK)], rows[0], ldsem[0])
    for k in range(_N_DCH):
        b = k % 2
        if k + 1 < _N_DCH:
            if k - 1 >= 0:
                h_sc[k - 1][0].wait()
                h_sc[k - 1][1].wait()
            h_load[k + 1] = pltpu.async_copy(
                fbf_hbm.at[pl.ds(tok0(k + 1), SC_CHUNK)], rows[(k + 1) % 2], ldsem[(k + 1) % 2]
            )
        pltpu.sync_copy(s1_hbm.at[pl.ds(tok0(k), SC_CHUNK)], idx1[b])
        pltpu.sync_copy(s2_hbm.at[pl.ds(tok0(k), SC_CHUNK)], idx2[b])
        h_load[k].wait()
        h_sc[k] = (
            pltpu.async_copy(rows[b], xg_hbm.at[idx1[b]], scsem[b]),
            pltpu.async_copy(rows[b], xg_hbm.at[idx2[b]], scsem[b]),
        )
    for k in (_N_DCH - 2, _N_DCH - 1):
        h_sc[k][0].wait()
        h_sc[k][1].wait()


def _dispatch(fbf, s1, s2):
    mesh = plsc.VectorSubcoreMesh(core_axis_name="c", subcore_axis_name="s")
    k = pl.kernel(
        _dispatch_body,
        out_type=jax.ShapeDtypeStruct((XG_ROWS, D_MODEL), jnp.float32),
        mesh=mesh,
        scratch_types=[
            [pltpu.VMEM((SC_CHUNK,), jnp.int32)] * 2,
            [pltpu.VMEM((SC_CHUNK,), jnp.int32)] * 2,
            [pltpu.VMEM((SC_CHUNK, D_MODEL), jnp.float32)] * 2,
            [pltpu.SemaphoreType.DMA] * 2,
            [pltpu.SemaphoreType.DMA] * 2,
        ],
    )
    return k(fbf, s1, s2)


def _ffn_body(xg_ref, w1_ref, b1_ref, w2_ref, b2_ref, out_ref, xbf):
    f = pl.program_id(1)

    @pl.when(f == 0)
    def _():
        xbf[...] = xg_ref[...].astype(jnp.bfloat16)

    w1b = w1_ref[0]
    w2b = w2_ref[0]
    b1v = b1_ref[0]
    b2v = b2_ref[0]
    for c in range(CAP // ROW_BLK):
        xa = xbf[pl.ds(c * ROW_BLK, ROW_BLK), :]
        h = lax.dot_general(
            xa, w1b, (((1,), (0,)), ((), ())), preferred_element_type=jnp.float32
        )
        h = h + b1v
        h = 0.5 * h * (1.0 + lax.erf(h * _RSQRT2))
        y = lax.dot_general(
            h.astype(jnp.bfloat16),
            w2b,
            (((1,), (0,)), ((), ())),
            preferred_element_type=jnp.float32,
        )

        @pl.when(f == 0)
        def _():
            out_ref[pl.ds(c * ROW_BLK, ROW_BLK), :] = y + b2v

        @pl.when(f != 0)
        def _():
            out_ref[pl.ds(c * ROW_BLK, ROW_BLK), :] += y


def _ffn(xg, w1, b1, w2, b2):
    return pl.pallas_call(
        _ffn_body,
        grid=(N_EXP, N_FF_BLKS),
        in_specs=[
            pl.BlockSpec((CAP, D_MODEL), lambda e, f: (e, 0)),
            pl.BlockSpec((1, D_MODEL, FF_BLK), lambda e, f: (e, 0, f)),
            pl.BlockSpec((1, 1, FF_BLK), lambda e, f: (e, 0, f)),
            pl.BlockSpec((1, FF_BLK, D_MODEL), lambda e, f: (e, f, 0)),
            pl.BlockSpec((1, 1, D_MODEL), lambda e, f: (e, 0, 0)),
        ],
        out_specs=pl.BlockSpec((CAP, D_MODEL), lambda e, f: (e, 0)),
        out_shape=jax.ShapeDtypeStruct((XG_ROWS, D_MODEL), jnp.float32),
        scratch_shapes=[pltpu.VMEM((CAP, D_MODEL), jnp.bfloat16)],
    )(xg, w1, b1.reshape(N_EXP, 1, D_FF), w2, b2.reshape(N_EXP, 1, D_MODEL))


CMB_CH = 16
_N_CCH = (N_TOK // 32) // CMB_CH


def _gather_body(out_hbm, s1_hbm, s2_hbm, g1_hbm, g2_hbm,
                 i1, i2, rows1, rows2, gsem, wsem):
    wid = lax.axis_index("c") * 16 + lax.axis_index("s")
    base = wid * (N_TOK // 32)

    def tok0(k):
        return base + k * CMB_CH

    def start_gather(k):
        b = k % 2
        pltpu.sync_copy(s1_hbm.at[pl.ds(tok0(k), CMB_CH)], i1[b])
        pltpu.sync_copy(s2_hbm.at[pl.ds(tok0(k), CMB_CH)], i2[b])
        return (
            pltpu.async_copy(out_hbm.at[i1[b]], rows1[b], gsem[b]),
            pltpu.async_copy(out_hbm.at[i2[b]], rows2[b], gsem[b]),
        )

    h_g = {0: start_gather(0)}
    h_w = {}
    for k in range(_N_CCH):
        b = k % 2
        h_g[k][0].wait()
        h_g[k][1].wait()
        if k - 2 >= 0:
            h_w[k - 2][0].wait()
            h_w[k - 2][1].wait()
        h_w[k] = (
            pltpu.async_copy(rows1[b], g1_hbm.at[pl.ds(tok0(k), CMB_CH)], wsem[b]),
            pltpu.async_copy(rows2[b], g2_hbm.at[pl.ds(tok0(k), CMB_CH)], wsem[b]),
        )
        if k + 1 < _N_CCH:
            h_g[k + 1] = start_gather(k + 1)
    for k in (_N_CCH - 2, _N_CCH - 1):
        h_w[k][0].wait()
        h_w[k][1].wait()


def _gather2(out_ffn, s1, s2):
    mesh = plsc.VectorSubcoreMesh(core_axis_name="c", subcore_axis_name="s")
    k = pl.kernel(
        _gather_body,
        out_type=(
            jax.ShapeDtypeStruct((N_TOK, D_MODEL), jnp.float32),
            jax.ShapeDtypeStruct((N_TOK, D_MODEL), jnp.float32),
        ),
        mesh=mesh,
        scratch_types=[
            [pltpu.VMEM((CMB_CH,), jnp.int32)] * 2,
            [pltpu.VMEM((CMB_CH,), jnp.int32)] * 2,
            [pltpu.VMEM((CMB_CH, D_MODEL), jnp.float32)] * 2,
            [pltpu.VMEM((CMB_CH, D_MODEL), jnp.float32)] * 2,
            [pltpu.SemaphoreType.DMA] * 2,
            [pltpu.SemaphoreType.DMA] * 2,
        ],
    )
    return k(out_ffn, s1, s2)


def _combine_body(g1_ref, g2_ref, w1_ref, w2_ref, o_ref):
    w1v = w1_ref[:, :1]
    w2v = w2_ref[:, :1]
    a = jnp.where(w1v > 0, g1_ref[...], 0.0) * w1v
    b = jnp.where(w2v > 0, g2_ref[...], 0.0) * w2v
    o_ref[...] = a + b


def _combine(g1, g2, w1x, w2x):
    return pl.pallas_call(
        _combine_body,
        grid=(N_TOK_BLKS,),
        in_specs=[
            pl.BlockSpec((TOK_BLK, D_MODEL), lambda i: (i, 0)),
            pl.BlockSpec((TOK_BLK, D_MODEL), lambda i: (i, 0)),
            pl.BlockSpec((TOK_BLK, 16), lambda i: (i, 0)),
            pl.BlockSpec((TOK_BLK, 16), lambda i: (i, 0)),
        ],
        out_specs=pl.BlockSpec((TOK_BLK, D_MODEL), lambda i: (i, 0)),
        out_shape=jax.ShapeDtypeStruct((N_TOK, D_MODEL), jnp.float32),
    )(g1, g2, w1x, w2x)


def kernel(hidden_states, Wg, W1, b1, W2, b2):
    bh, th, d = hidden_states.shape
    flat = hidden_states.reshape(bh * th, d)
    s1, s2, w1x, w2x = _router(flat, Wg)
    s1f = s1.reshape(N_TOK)
    s2f = s2.reshape(N_TOK)
    xg = _dispatch(flat, s1f, s2f)
    out_ffn = _ffn(xg, W1.astype(jnp.bfloat16), b1, W2.astype(jnp.bfloat16), b2)
    g1, g2 = _gather2(out_ffn, s1f, s2f)
    final = _combine(g1, g2, w1x, w2x)
    aux_loss = jnp.asarray(0.0, dtype=jnp.float32)
    return final.reshape(bh, th, d), aux_loss


# speedup vs baseline: 1.3780x; 1.3780x over previous
import functools
import math

import jax
import jax.numpy as jnp
from jax import lax
from jax.experimental import pallas as pl
from jax.experimental.pallas import tpu as pltpu
from jax.experimental.pallas import tpu_sc as plsc

N_TOK = 8192
D_MODEL = 1024
D_FF = 4096
N_EXP = 8
TOPK = 2
CAP = 2048
DUMMY = N_EXP * CAP
XG_ROWS = N_EXP * CAP + CAP

TOK_BLK = 256
N_TOK_BLKS = N_TOK // TOK_BLK
FF_BLK = 1024
N_FF_BLKS = D_FF // FF_BLK
ROW_BLK = 512
D_HALF = D_MODEL // 2


def _pack_rows(x32):
    lo = lax.bitcast_convert_type(x32[:, :D_HALF].astype(jnp.bfloat16), jnp.uint16)
    hi = lax.bitcast_convert_type(x32[:, D_HALF:].astype(jnp.bfloat16), jnp.uint16)
    u = lo.astype(jnp.uint32) | (hi.astype(jnp.uint32) << 16)
    return lax.bitcast_convert_type(u, jnp.float32)


def _unpack_rows(p32):
    u = lax.bitcast_convert_type(p32, jnp.uint32)
    lo = lax.bitcast_convert_type((u & 0xFFFF).astype(jnp.uint16), jnp.bfloat16)
    hi = lax.bitcast_convert_type((u >> 16).astype(jnp.uint16), jnp.bfloat16)
    return jnp.concatenate([lo, hi], axis=1)

SC_CHUNK = 64
_RSQRT2 = 1.0 / math.sqrt(2.0)


def _router_body(x_ref, wg_ref, s1_ref, s2_ref, w1_ref, w2_ref, fp_ref, carry_ref):
    i = pl.program_id(0)

    @pl.when(i == 0)
    def _():
        carry_ref[...] = jnp.zeros((1, N_EXP), jnp.float32)

    x = x_ref[...]
    fp_ref[...] = _pack_rows(x)
    wg = wg_ref[...]
    logits = lax.dot_general(
        x, wg, (((1,), (1,)), ((), ())), preferred_element_type=jnp.float32
    )
    z = logits - jnp.max(logits, axis=1, keepdims=True)
    p = jnp.exp(z)
    gates = p / jnp.sum(p, axis=1, keepdims=True)

    e_iota = lax.broadcasted_iota(jnp.int32, (TOK_BLK, N_EXP), 1)
    v1 = jnp.max(gates, axis=1, keepdims=True)
    i1 = jnp.min(jnp.where(gates == v1, e_iota, N_EXP), axis=1, keepdims=True)
    oh1 = e_iota == i1
    g2 = jnp.where(oh1, -1.0, gates)
    v2 = jnp.max(g2, axis=1, keepdims=True)
    i2 = jnp.min(jnp.where(g2 == v2, e_iota, N_EXP), axis=1, keepdims=True)
    oh2 = e_iota == i2

    mf = (oh1 | oh2).astype(jnp.float32)
    row = lax.broadcasted_iota(jnp.int32, (TOK_BLK, TOK_BLK), 0)
    col = lax.broadcasted_iota(jnp.int32, (TOK_BLK, TOK_BLK), 1)
    trif = (row >= col).astype(jnp.float32)
    incl = lax.dot_general(
        trif, mf, (((1,), (0,)), ((), ())), preferred_element_type=jnp.float32
    )
    pos = carry_ref[...] + incl - 1.0
    keepf = jnp.where(pos < float(CAP), mf, 0.0)
    gsf = e_iota.astype(jnp.float32) * float(CAP) + pos
    gs = jnp.where(keepf > 0, gsf, float(DUMMY))

    slot1 = jnp.sum(jnp.where(oh1, gs, 0.0), axis=1, keepdims=True)
    slot2 = jnp.sum(jnp.where(oh2, gs, 0.0), axis=1, keepdims=True)
    k1 = jnp.sum(jnp.where(oh1, keepf, 0.0), axis=1, keepdims=True)
    k2 = jnp.sum(jnp.where(oh2, keepf, 0.0), axis=1, keepdims=True)

    s1_ref[...] = slot1.astype(jnp.int32)
    s2_ref[...] = slot2.astype(jnp.int32)
    w1_ref[...] = jnp.broadcast_to(v1 * k1, (TOK_BLK, 16))
    w2_ref[...] = jnp.broadcast_to(v2 * k2, (TOK_BLK, 16))
    carry_ref[...] = carry_ref[...] + jnp.sum(mf, axis=0, keepdims=True)


def _router(flat, wg):
    return pl.pallas_call(
        _router_body,
        grid=(N_TOK_BLKS,),
        in_specs=[
            pl.BlockSpec((TOK_BLK, D_MODEL), lambda i: (i, 0)),
            pl.BlockSpec((N_EXP, D_MODEL), lambda i: (0, 0)),
        ],
        out_specs=[
            pl.BlockSpec((TOK_BLK, 1), lambda i: (i, 0)),
            pl.BlockSpec((TOK_BLK, 1), lambda i: (i, 0)),
            pl.BlockSpec((TOK_BLK, 16), lambda i: (i, 0)),
            pl.BlockSpec((TOK_BLK, 16), lambda i: (i, 0)),
            pl.BlockSpec((TOK_BLK, D_HALF), lambda i: (i, 0)),
        ],
        out_shape=[
            jax.ShapeDtypeStruct((N_TOK, 1), jnp.int32),
            jax.ShapeDtypeStruct((N_TOK, 1), jnp.int32),
            jax.ShapeDtypeStruct((N_TOK, 16), jnp.float32),
            jax.ShapeDtypeStruct((N_TOK, 16), jnp.float32),
            jax.ShapeDtypeStruct((N_TOK, D_HALF), jnp.float32),
        ],
        scratch_shapes=[pltpu.VMEM((1, N_EXP), jnp.float32)],
    )(flat, wg)


_N_DCH = (N_TOK // 32) // SC_CHUNK


def _dispatch_body(fbf_hbm, s1_hbm, s2_hbm, xg_hbm, idx1, idx2, rows, ldsem, scsem):
    wid = lax.axis_index("c") * 16 + lax.axis_index("s")
    base = wid * (N_TOK // 32)

    def tok0(k):
        return base + k * SC_CHUNK

    h_load = {}
    h_sc = {}
    h_load[0] = pltpu.async_copy(fbf_hbm.at[pl.ds(tok0(0), SC_CHUNK)], rows[0], ldsem[0])
    for k in range(_N_DCH):
        b = k % 2
        if k + 1 < _N_DCH:
            if k - 1 >= 0:
                h_sc[k - 1][0].wait()
                h_sc[k - 1][1].wait()
            h_load[k + 1] = pltpu.async_copy(
                fbf_hbm.at[pl.ds(tok0(k + 1), SC_CHUNK)], rows[(k + 1) % 2], ldsem[(k + 1) % 2]
            )
        pltpu.sync_copy(s1_hbm.at[pl.ds(tok0(k), SC_CHUNK)], idx1[b])
        pltpu.sync_copy(s2_hbm.at[pl.ds(tok0(k), SC_CHUNK)], idx2[b])
        h_load[k].wait()
        h_sc[k] = (
            pltpu.async_copy(rows[b], xg_hbm.at[idx1[b]], scsem[b]),
            pltpu.async_copy(rows[b], xg_hbm.at[idx2[b]], scsem[b]),
        )
    for k in (_N_DCH - 2, _N_DCH - 1):
        h_sc[k][0].wait()
        h_sc[k][1].wait()


def _dispatch(fbf, s1, s2):
    mesh = plsc.VectorSubcoreMesh(core_axis_name="c", subcore_axis_name="s")
    k = pl.kernel(
        _dispatch_body,
        out_type=jax.ShapeDtypeStruct((XG_ROWS, D_HALF), jnp.float32),
        mesh=mesh,
        scratch_types=[
            [pltpu.VMEM((SC_CHUNK,), jnp.int32)] * 2,
            [pltpu.VMEM((SC_CHUNK,), jnp.int32)] * 2,
            [pltpu.VMEM((SC_CHUNK, D_HALF), jnp.float32)] * 2,
            [pltpu.SemaphoreType.DMA] * 2,
            [pltpu.SemaphoreType.DMA] * 2,
        ],
    )
    return k(fbf, s1, s2)


def _ffn_body(xg_ref, w1_ref, b1_ref, w2_ref, b2_ref, out_ref, acc):
    f = pl.program_id(1)

    w1b = w1_ref[0].astype(jnp.bfloat16)
    w2b = w2_ref[0].astype(jnp.bfloat16)
    b1v = b1_ref[0]
    b2v = b2_ref[0]
    for c in range(CAP // ROW_BLK):
        xa = _unpack_rows(xg_ref[pl.ds(c * ROW_BLK, ROW_BLK), :])
        h = lax.dot_general(
            xa, w1b, (((1,), (0,)), ((), ())), preferred_element_type=jnp.float32
        )
        h = h + b1v
        h = 0.5 * h * (1.0 + lax.erf(h * _RSQRT2))
        y = lax.dot_general(
            h.astype(jnp.bfloat16),
            w2b,
            (((1,), (0,)), ((), ())),
            preferred_element_type=jnp.float32,
        )
        sl = pl.ds(c * ROW_BLK, ROW_BLK)

        @pl.when(f == 0)
        def _():
            acc[sl, :] = y + b2v

        @pl.when(f != 0)
        def _():
            acc[sl, :] += y

        @pl.when(f == N_FF_BLKS - 1)
        def _():
            out_ref[sl, :] = _pack_rows(acc[sl, :])


def _ffn(xg, w1, b1, w2, b2):
    return pl.pallas_call(
        _ffn_body,
        grid=(N_EXP, N_FF_BLKS),
        in_specs=[
            pl.BlockSpec((CAP, D_HALF), lambda e, f: (e, 0)),
            pl.BlockSpec((1, D_MODEL, FF_BLK), lambda e, f: (e, 0, f)),
            pl.BlockSpec((1, 1, FF_BLK), lambda e, f: (e, 0, f)),
            pl.BlockSpec((1, FF_BLK, D_MODEL), lambda e, f: (e, f, 0)),
            pl.BlockSpec((1, 1, D_MODEL), lambda e, f: (e, 0, 0)),
        ],
        out_specs=pl.BlockSpec((CAP, D_HALF), lambda e, f: (e, 0)),
        out_shape=jax.ShapeDtypeStruct((XG_ROWS, D_HALF), jnp.float32),
        scratch_shapes=[pltpu.VMEM((CAP, D_MODEL), jnp.float32)],
    )(xg, w1, b1.reshape(N_EXP, 1, D_FF), w2, b2.reshape(N_EXP, 1, D_MODEL))


CMB_CH = 32
_N_CCH = (N_TOK // 32) // CMB_CH


def _gather_body(out_hbm, s1_hbm, s2_hbm, g1_hbm, g2_hbm,
                 i1, i2, rows1, rows2, gsem, wsem):
    wid = lax.axis_index("c") * 16 + lax.axis_index("s")
    base = wid * (N_TOK // 32)

    def tok0(k):
        return base + k * CMB_CH

    def start_gather(k):
        b = k % 2
        pltpu.sync_copy(s1_hbm.at[pl.ds(tok0(k), CMB_CH)], i1[b])
        pltpu.sync_copy(s2_hbm.at[pl.ds(tok0(k), CMB_CH)], i2[b])
        return (
            pltpu.async_copy(out_hbm.at[i1[b]], rows1[b], gsem[b]),
            pltpu.async_copy(out_hbm.at[i2[b]], rows2[b], gsem[b]),
        )

    h_g = {0: start_gather(0)}
    h_w = {}
    for k in range(_N_CCH):
        b = k % 2
        h_g[k][0].wait()
        h_g[k][1].wait()
        if k - 2 >= 0:
            h_w[k - 2][0].wait()
            h_w[k - 2][1].wait()
        h_w[k] = (
            pltpu.async_copy(rows1[b], g1_hbm.at[pl.ds(tok0(k), CMB_CH)], wsem[b]),
            pltpu.async_copy(rows2[b], g2_hbm.at[pl.ds(tok0(k), CMB_CH)], wsem[b]),
        )
        if k + 1 < _N_CCH:
            h_g[k + 1] = start_gather(k + 1)
    for k in (_N_CCH - 2, _N_CCH - 1):
        h_w[k][0].wait()
        h_w[k][1].wait()


def _gather2(out_ffn, s1, s2):
    mesh = plsc.VectorSubcoreMesh(core_axis_name="c", subcore_axis_name="s")
    k = pl.kernel(
        _gather_body,
        out_type=(
            jax.ShapeDtypeStruct((N_TOK, D_HALF), jnp.float32),
            jax.ShapeDtypeStruct((N_TOK, D_HALF), jnp.float32),
        ),
        mesh=mesh,
        scratch_types=[
            [pltpu.VMEM((CMB_CH,), jnp.int32)] * 2,
            [pltpu.VMEM((CMB_CH,), jnp.int32)] * 2,
            [pltpu.VMEM((CMB_CH, D_HALF), jnp.float32)] * 2,
            [pltpu.VMEM((CMB_CH, D_HALF), jnp.float32)] * 2,
            [pltpu.SemaphoreType.DMA] * 2,
            [pltpu.SemaphoreType.DMA] * 2,
        ],
    )
    return k(out_ffn, s1, s2)


def _combine_body(g1_ref, g2_ref, w1_ref, w2_ref, o_ref):
    w1v = w1_ref[:, :1]
    w2v = w2_ref[:, :1]
    y1 = _unpack_rows(g1_ref[...]).astype(jnp.float32)
    y2 = _unpack_rows(g2_ref[...]).astype(jnp.float32)
    a = jnp.where(w1v > 0, y1, 0.0) * w1v
    b = jnp.where(w2v > 0, y2, 0.0) * w2v
    o_ref[...] = a + b


def _combine(g1, g2, w1x, w2x):
    return pl.pallas_call(
        _combine_body,
        grid=(N_TOK_BLKS,),
        in_specs=[
            pl.BlockSpec((TOK_BLK, D_HALF), lambda i: (i, 0)),
            pl.BlockSpec((TOK_BLK, D_HALF), lambda i: (i, 0)),
            pl.BlockSpec((TOK_BLK, 16), lambda i: (i, 0)),
            pl.BlockSpec((TOK_BLK, 16), lambda i: (i, 0)),
        ],
        out_specs=pl.BlockSpec((TOK_BLK, D_MODEL), lambda i: (i, 0)),
        out_shape=jax.ShapeDtypeStruct((N_TOK, D_MODEL), jnp.float32),
    )(g1, g2, w1x, w2x)


def kernel(hidden_states, Wg, W1, b1, W2, b2):
    bh, th, d = hidden_states.shape
    flat = hidden_states.reshape(bh * th, d)
    s1, s2, w1x, w2x, fpk = _router(flat, Wg)
    s1f = s1.reshape(N_TOK)
    s2f = s2.reshape(N_TOK)
    xg = _dispatch(fpk, s1f, s2f)
    out_ffn = _ffn(xg, W1, b1, W2, b2)
    g1, g2 = _gather2(out_ffn, s1f, s2f)
    final = _combine(g1, g2, w1x, w2x)
    aux_loss = jnp.asarray(0.0, dtype=jnp.float32)
    return final.reshape(bh, th, d), aux_loss


# speedup vs baseline: 1.3788x; 1.0006x over previous
import functools
import math

import jax
import jax.numpy as jnp
from jax import lax
from jax.experimental import pallas as pl
from jax.experimental.pallas import tpu as pltpu
from jax.experimental.pallas import tpu_sc as plsc

N_TOK = 8192
D_MODEL = 1024
D_FF = 4096
N_EXP = 8
TOPK = 2
CAP = 2048
DUMMY = N_EXP * CAP
XG_ROWS = N_EXP * CAP + CAP

TOK_BLK = 256
N_TOK_BLKS = N_TOK // TOK_BLK
FF_BLK = 1024
N_FF_BLKS = D_FF // FF_BLK
ROW_BLK = 512
D_HALF = D_MODEL // 2


def _pack_rows(x32):
    lo = lax.bitcast_convert_type(x32[:, :D_HALF].astype(jnp.bfloat16), jnp.uint16)
    hi = lax.bitcast_convert_type(x32[:, D_HALF:].astype(jnp.bfloat16), jnp.uint16)
    u = lo.astype(jnp.uint32) | (hi.astype(jnp.uint32) << 16)
    return lax.bitcast_convert_type(u, jnp.float32)


def _unpack_rows(p32):
    u = lax.bitcast_convert_type(p32, jnp.uint32)
    lo = lax.bitcast_convert_type((u & 0xFFFF).astype(jnp.uint16), jnp.bfloat16)
    hi = lax.bitcast_convert_type((u >> 16).astype(jnp.uint16), jnp.bfloat16)
    return jnp.concatenate([lo, hi], axis=1)

SC_CHUNK = 64
_RSQRT2 = 1.0 / math.sqrt(2.0)


def _router_body(x_ref, wg_ref, s1_ref, s2_ref, w1_ref, w2_ref, fp_ref, carry_ref):
    i = pl.program_id(0)

    @pl.when(i == 0)
    def _():
        carry_ref[...] = jnp.zeros((1, N_EXP), jnp.float32)

    x = x_ref[...]
    fp_ref[...] = _pack_rows(x)
    wg = wg_ref[...]
    logits = lax.dot_general(
        x, wg, (((1,), (1,)), ((), ())), preferred_element_type=jnp.float32
    )
    z = logits - jnp.max(logits, axis=1, keepdims=True)
    p = jnp.exp(z)
    gates = p / jnp.sum(p, axis=1, keepdims=True)

    e_iota = lax.broadcasted_iota(jnp.int32, (TOK_BLK, N_EXP), 1)
    v1 = jnp.max(gates, axis=1, keepdims=True)
    i1 = jnp.min(jnp.where(gates == v1, e_iota, N_EXP), axis=1, keepdims=True)
    oh1 = e_iota == i1
    g2 = jnp.where(oh1, -1.0, gates)
    v2 = jnp.max(g2, axis=1, keepdims=True)
    i2 = jnp.min(jnp.where(g2 == v2, e_iota, N_EXP), axis=1, keepdims=True)
    oh2 = e_iota == i2

    mf = (oh1 | oh2).astype(jnp.float32)
    row = lax.broadcasted_iota(jnp.int32, (TOK_BLK, TOK_BLK), 0)
    col = lax.broadcasted_iota(jnp.int32, (TOK_BLK, TOK_BLK), 1)
    trif = (row >= col).astype(jnp.float32)
    incl = lax.dot_general(
        trif, mf, (((1,), (0,)), ((), ())), preferred_element_type=jnp.float32
    )
    pos = carry_ref[...] + incl - 1.0
    keepf = jnp.where(pos < float(CAP), mf, 0.0)
    gsf = e_iota.astype(jnp.float32) * float(CAP) + pos
    gs = jnp.where(keepf > 0, gsf, float(DUMMY))

    slot1 = jnp.sum(jnp.where(oh1, gs, 0.0), axis=1, keepdims=True)
    slot2 = jnp.sum(jnp.where(oh2, gs, 0.0), axis=1, keepdims=True)
    k1 = jnp.sum(jnp.where(oh1, keepf, 0.0), axis=1, keepdims=True)
    k2 = jnp.sum(jnp.where(oh2, keepf, 0.0), axis=1, keepdims=True)

    s1_ref[...] = slot1.astype(jnp.int32)
    s2_ref[...] = slot2.astype(jnp.int32)
    w1_ref[...] = jnp.broadcast_to(v1 * k1, (TOK_BLK, 16))
    w2_ref[...] = jnp.broadcast_to(v2 * k2, (TOK_BLK, 16))
    carry_ref[...] = carry_ref[...] + jnp.sum(mf, axis=0, keepdims=True)


def _router(flat, wg):
    return pl.pallas_call(
        _router_body,
        grid=(N_TOK_BLKS,),
        in_specs=[
            pl.BlockSpec((TOK_BLK, D_MODEL), lambda i: (i, 0)),
            pl.BlockSpec((N_EXP, D_MODEL), lambda i: (0, 0)),
        ],
        out_specs=[
            pl.BlockSpec((TOK_BLK, 1), lambda i: (i, 0)),
            pl.BlockSpec((TOK_BLK, 1), lambda i: (i, 0)),
            pl.BlockSpec((TOK_BLK, 16), lambda i: (i, 0)),
            pl.BlockSpec((TOK_BLK, 16), lambda i: (i, 0)),
            pl.BlockSpec((TOK_BLK, D_HALF), lambda i: (i, 0)),
        ],
        out_shape=[
            jax.ShapeDtypeStruct((N_TOK, 1), jnp.int32),
            jax.ShapeDtypeStruct((N_TOK, 1), jnp.int32),
            jax.ShapeDtypeStruct((N_TOK, 16), jnp.float32),
            jax.ShapeDtypeStruct((N_TOK, 16), jnp.float32),
            jax.ShapeDtypeStruct((N_TOK, D_HALF), jnp.float32),
        ],
        scratch_shapes=[pltpu.VMEM((1, N_EXP), jnp.float32)],
    )(flat, wg)


_N_DCH = (N_TOK // 32) // SC_CHUNK


def _dispatch_body(fbf_hbm, s1_hbm, s2_hbm, xg_hbm, idx1, idx2, rows, ldsem, scsem):
    wid = lax.axis_index("c") * 16 + lax.axis_index("s")
    base = wid * (N_TOK // 32)

    def tok0(k):
        return base + k * SC_CHUNK

    h_load = {}
    h_sc = {}
    h_load[0] = pltpu.async_copy(fbf_hbm.at[pl.ds(tok0(0), SC_CHUNK)], rows[0], ldsem[0])
    for k in range(_N_DCH):
        b = k % 2
        if k + 1 < _N_DCH:
            if k - 1 >= 0:
                h_sc[k - 1][0].wait()
                h_sc[k - 1][1].wait()
            h_load[k + 1] = pltpu.async_copy(
                fbf_hbm.at[pl.ds(tok0(k + 1), SC_CHUNK)], rows[(k + 1) % 2], ldsem[(k + 1) % 2]
            )
        pltpu.sync_copy(s1_hbm.at[pl.ds(tok0(k), SC_CHUNK)], idx1[b])
        pltpu.sync_copy(s2_hbm.at[pl.ds(tok0(k), SC_CHUNK)], idx2[b])
        h_load[k].wait()
        h_sc[k] = (
            pltpu.async_copy(rows[b], xg_hbm.at[idx1[b]], scsem[b]),
            pltpu.async_copy(rows[b], xg_hbm.at[idx2[b]], scsem[b]),
        )
    for k in (_N_DCH - 2, _N_DCH - 1):
        h_sc[k][0].wait()
        h_sc[k][1].wait()


def _dispatch(fbf, s1, s2):
    mesh = plsc.VectorSubcoreMesh(core_axis_name="c", subcore_axis_name="s")
    k = pl.kernel(
        _dispatch_body,
        out_type=jax.ShapeDtypeStruct((XG_ROWS, D_HALF), jnp.float32),
        mesh=mesh,
        scratch_types=[
            [pltpu.VMEM((SC_CHUNK,), jnp.int32)] * 2,
            [pltpu.VMEM((SC_CHUNK,), jnp.int32)] * 2,
            [pltpu.VMEM((SC_CHUNK, D_HALF), jnp.float32)] * 2,
            [pltpu.SemaphoreType.DMA] * 2,
            [pltpu.SemaphoreType.DMA] * 2,
        ],
    )
    return k(fbf, s1, s2)


def _ffn_body(xg_ref, w1_ref, b1_ref, w2_ref, b2_ref, out_ref, acc):
    f = pl.program_id(1)

    w1b = w1_ref[0].astype(jnp.bfloat16)
    w2b = w2_ref[0].astype(jnp.bfloat16)
    b1v = b1_ref[0]
    b2v = b2_ref[0]
    for c in range(CAP // ROW_BLK):
        u = lax.bitcast_convert_type(xg_ref[pl.ds(c * ROW_BLK, ROW_BLK), :], jnp.uint32)
        xlo = lax.bitcast_convert_type((u & 0xFFFF).astype(jnp.uint16), jnp.bfloat16)
        xhi = lax.bitcast_convert_type((u >> 16).astype(jnp.uint16), jnp.bfloat16)
        h = lax.dot_general(
            xlo, w1b[:D_HALF], (((1,), (0,)), ((), ())), preferred_element_type=jnp.float32
        ) + lax.dot_general(
            xhi, w1b[D_HALF:], (((1,), (0,)), ((), ())), preferred_element_type=jnp.float32
        )
        h = h + b1v
        h = 0.5 * h * (1.0 + lax.erf(h * _RSQRT2))
        y = lax.dot_general(
            h.astype(jnp.bfloat16),
            w2b,
            (((1,), (0,)), ((), ())),
            preferred_element_type=jnp.float32,
        )
        sl = pl.ds(c * ROW_BLK, ROW_BLK)

        @pl.when(f == 0)
        def _():
            acc[sl, :] = y + b2v

        @pl.when(f != 0)
        def _():
            acc[sl, :] += y

        @pl.when(f == N_FF_BLKS - 1)
        def _():
            out_ref[sl, :] = _pack_rows(acc[sl, :])


def _ffn(xg, w1, b1, w2, b2):
    return pl.pallas_call(
        _ffn_body,
        grid=(N_EXP, N_FF_BLKS),
        in_specs=[
            pl.BlockSpec((CAP, D_HALF), lambda e, f: (e, 0)),
            pl.BlockSpec((1, D_MODEL, FF_BLK), lambda e, f: (e, 0, f)),
            pl.BlockSpec((1, 1, FF_BLK), lambda e, f: (e, 0, f)),
            pl.BlockSpec((1, FF_BLK, D_MODEL), lambda e, f: (e, f, 0)),
            pl.BlockSpec((1, 1, D_MODEL), lambda e, f: (e, 0, 0)),
        ],
        out_specs=pl.BlockSpec((CAP, D_HALF), lambda e, f: (e, 0)),
        out_shape=jax.ShapeDtypeStruct((XG_ROWS, D_HALF), jnp.float32),
        scratch_shapes=[pltpu.VMEM((CAP, D_MODEL), jnp.float32)],
    )(xg, w1, b1.reshape(N_EXP, 1, D_FF), w2, b2.reshape(N_EXP, 1, D_MODEL))


CMB_CH = 32
_N_CCH = (N_TOK // 32) // CMB_CH


def _gather_body(out_hbm, s1_hbm, s2_hbm, g1_hbm, g2_hbm,
                 i1, i2, rows1, rows2, gsem, wsem):
    wid = lax.axis_index("c") * 16 + lax.axis_index("s")
    base = wid * (N_TOK // 32)

    def tok0(k):
        return base + k * CMB_CH

    def start_gather(k):
        b = k % 2
        pltpu.sync_copy(s1_hbm.at[pl.ds(tok0(k), CMB_CH)], i1[b])
        pltpu.sync_copy(s2_hbm.at[pl.ds(tok0(k), CMB_CH)], i2[b])
        return (
            pltpu.async_copy(out_hbm.at[i1[b]], rows1[b], gsem[b]),
            pltpu.async_copy(out_hbm.at[i2[b]], rows2[b], gsem[b]),
        )

    h_g = {0: start_gather(0)}
    h_w = {}
    for k in range(_N_CCH):
        b = k % 2
        h_g[k][0].wait()
        h_g[k][1].wait()
        if k - 2 >= 0:
            h_w[k - 2][0].wait()
            h_w[k - 2][1].wait()
        h_w[k] = (
            pltpu.async_copy(rows1[b], g1_hbm.at[pl.ds(tok0(k), CMB_CH)], wsem[b]),
            pltpu.async_copy(rows2[b], g2_hbm.at[pl.ds(tok0(k), CMB_CH)], wsem[b]),
        )
        if k + 1 < _N_CCH:
            h_g[k + 1] = start_gather(k + 1)
    for k in (_N_CCH - 2, _N_CCH - 1):
        h_w[k][0].wait()
        h_w[k][1].wait()


def _gather2(out_ffn, s1, s2):
    mesh = plsc.VectorSubcoreMesh(core_axis_name="c", subcore_axis_name="s")
    k = pl.kernel(
        _gather_body,
        out_type=(
            jax.ShapeDtypeStruct((N_TOK, D_HALF), jnp.float32),
            jax.ShapeDtypeStruct((N_TOK, D_HALF), jnp.float32),
        ),
        mesh=mesh,
        scratch_types=[
            [pltpu.VMEM((CMB_CH,), jnp.int32)] * 2,
            [pltpu.VMEM((CMB_CH,), jnp.int32)] * 2,
            [pltpu.VMEM((CMB_CH, D_HALF), jnp.float32)] * 2,
            [pltpu.VMEM((CMB_CH, D_HALF), jnp.float32)] * 2,
            [pltpu.SemaphoreType.DMA] * 2,
            [pltpu.SemaphoreType.DMA] * 2,
        ],
    )
    return k(out_ffn, s1, s2)


def _combine_body(g1_ref, g2_ref, w1_ref, w2_ref, o_ref):
    w1v = w1_ref[:, :1]
    w2v = w2_ref[:, :1]
    y1 = _unpack_rows(g1_ref[...]).astype(jnp.float32)
    y2 = _unpack_rows(g2_ref[...]).astype(jnp.float32)
    a = jnp.where(w1v > 0, y1, 0.0) * w1v
    b = jnp.where(w2v > 0, y2, 0.0) * w2v
    o_ref[...] = a + b


def _combine(g1, g2, w1x, w2x):
    return pl.pallas_call(
        _combine_body,
        grid=(N_TOK_BLKS,),
        in_specs=[
            pl.BlockSpec((TOK_BLK, D_HALF), lambda i: (i, 0)),
            pl.BlockSpec((TOK_BLK, D_HALF), lambda i: (i, 0)),
            pl.BlockSpec((TOK_BLK, 16), lambda i: (i, 0)),
            pl.BlockSpec((TOK_BLK, 16), lambda i: (i, 0)),
        ],
        out_specs=pl.BlockSpec((TOK_BLK, D_MODEL), lambda i: (i, 0)),
        out_shape=jax.ShapeDtypeStruct((N_TOK, D_MODEL), jnp.float32),
    )(g1, g2, w1x, w2x)


def kernel(hidden_states, Wg, W1, b1, W2, b2):
    bh, th, d = hidden_states.shape
    flat = hidden_states.reshape(bh * th, d)
    s1, s2, w1x, w2x, fpk = _router(flat, Wg)
    s1f = s1.reshape(N_TOK)
    s2f = s2.reshape(N_TOK)
    xg = _dispatch(fpk, s1f, s2f)
    out_ffn = _ffn(xg, W1, b1, W2, b2)
    g1, g2 = _gather2(out_ffn, s1f, s2f)
    final = _combine(g1, g2, w1x, w2x)
    aux_loss = jnp.asarray(0.0, dtype=jnp.float32)
    return final.reshape(bh, th, d), aux_loss


# speedup vs baseline: 1.4431x; 1.0466x over previous
import functools
import math

import jax
import jax.numpy as jnp
from jax import lax
from jax.experimental import pallas as pl
from jax.experimental.pallas import tpu as pltpu
from jax.experimental.pallas import tpu_sc as plsc

N_TOK = 8192
D_MODEL = 1024
D_FF = 4096
N_EXP = 8
TOPK = 2
CAP = 2048
DUMMY = N_EXP * CAP
XG_ROWS = N_EXP * CAP + CAP

TOK_BLK = 512
N_TOK_BLKS = N_TOK // TOK_BLK
FF_BLK = 1024
N_FF_BLKS = D_FF // FF_BLK
ROW_BLK = 512
D_HALF = D_MODEL // 2


def _pack_rows(x32):
    lo = lax.bitcast_convert_type(x32[:, :D_HALF].astype(jnp.bfloat16), jnp.uint16)
    hi = lax.bitcast_convert_type(x32[:, D_HALF:].astype(jnp.bfloat16), jnp.uint16)
    u = lo.astype(jnp.uint32) | (hi.astype(jnp.uint32) << 16)
    return lax.bitcast_convert_type(u, jnp.float32)


def _unpack_rows(p32):
    u = lax.bitcast_convert_type(p32, jnp.uint32)
    lo = lax.bitcast_convert_type((u & 0xFFFF).astype(jnp.uint16), jnp.bfloat16)
    hi = lax.bitcast_convert_type((u >> 16).astype(jnp.uint16), jnp.bfloat16)
    return jnp.concatenate([lo, hi], axis=1)

SC_CHUNK = 64
_RSQRT2 = 1.0 / math.sqrt(2.0)


def _router_body(x_ref, wg_ref, s1_ref, s2_ref, w1_ref, w2_ref, fp_ref, carry_ref):
    i = pl.program_id(0)

    @pl.when(i == 0)
    def _():
        carry_ref[...] = jnp.zeros((1, N_EXP), jnp.float32)

    x = x_ref[...]
    fp_ref[...] = _pack_rows(x)
    wg = wg_ref[...]
    logits = lax.dot_general(
        x, wg, (((1,), (1,)), ((), ())), preferred_element_type=jnp.float32
    )
    z = logits - jnp.max(logits, axis=1, keepdims=True)
    p = jnp.exp(z)
    gates = p / jnp.sum(p, axis=1, keepdims=True)

    e_iota = lax.broadcasted_iota(jnp.int32, (TOK_BLK, N_EXP), 1)
    v1 = jnp.max(gates, axis=1, keepdims=True)
    i1 = jnp.min(jnp.where(gates == v1, e_iota, N_EXP), axis=1, keepdims=True)
    oh1 = e_iota == i1
    g2 = jnp.where(oh1, -1.0, gates)
    v2 = jnp.max(g2, axis=1, keepdims=True)
    i2 = jnp.min(jnp.where(g2 == v2, e_iota, N_EXP), axis=1, keepdims=True)
    oh2 = e_iota == i2

    mf = (oh1 | oh2).astype(jnp.float32)
    row = lax.broadcasted_iota(jnp.int32, (TOK_BLK, TOK_BLK), 0)
    col = lax.broadcasted_iota(jnp.int32, (TOK_BLK, TOK_BLK), 1)
    trif = (row >= col).astype(jnp.float32)
    incl = lax.dot_general(
        trif, mf, (((1,), (0,)), ((), ())), preferred_element_type=jnp.float32
    )
    pos = carry_ref[...] + incl - 1.0
    keepf = jnp.where(pos < float(CAP), mf, 0.0)
    gsf = e_iota.astype(jnp.float32) * float(CAP) + pos
    gs = jnp.where(keepf > 0, gsf, float(DUMMY))

    slot1 = jnp.sum(jnp.where(oh1, gs, 0.0), axis=1, keepdims=True)
    slot2 = jnp.sum(jnp.where(oh2, gs, 0.0), axis=1, keepdims=True)
    k1 = jnp.sum(jnp.where(oh1, keepf, 0.0), axis=1, keepdims=True)
    k2 = jnp.sum(jnp.where(oh2, keepf, 0.0), axis=1, keepdims=True)

    s1_ref[...] = slot1.astype(jnp.int32)
    s2_ref[...] = slot2.astype(jnp.int32)
    w1_ref[...] = jnp.broadcast_to(v1 * k1, (TOK_BLK, 16))
    w2_ref[...] = jnp.broadcast_to(v2 * k2, (TOK_BLK, 16))
    carry_ref[...] = carry_ref[...] + jnp.sum(mf, axis=0, keepdims=True)


def _router(flat, wg):
    return pl.pallas_call(
        _router_body,
        grid=(N_TOK_BLKS,),
        in_specs=[
            pl.BlockSpec((TOK_BLK, D_MODEL), lambda i: (i, 0)),
            pl.BlockSpec((N_EXP, D_MODEL), lambda i: (0, 0)),
        ],
        out_specs=[
            pl.BlockSpec((TOK_BLK, 1), lambda i: (i, 0)),
            pl.BlockSpec((TOK_BLK, 1), lambda i: (i, 0)),
            pl.BlockSpec((TOK_BLK, 16), lambda i: (i, 0)),
            pl.BlockSpec((TOK_BLK, 16), lambda i: (i, 0)),
            pl.BlockSpec((TOK_BLK, D_HALF), lambda i: (i, 0)),
        ],
        out_shape=[
            jax.ShapeDtypeStruct((N_TOK, 1), jnp.int32),
            jax.ShapeDtypeStruct((N_TOK, 1), jnp.int32),
            jax.ShapeDtypeStruct((N_TOK, 16), jnp.float32),
            jax.ShapeDtypeStruct((N_TOK, 16), jnp.float32),
            jax.ShapeDtypeStruct((N_TOK, D_HALF), jnp.float32),
        ],
        scratch_shapes=[pltpu.VMEM((1, N_EXP), jnp.float32)],
    )(flat, wg)


_N_DCH = (N_TOK // 32) // SC_CHUNK


def _dispatch_body(fbf_hbm, s1_hbm, s2_hbm, xg_hbm, idx1, idx2, rows, ldsem, scsem):
    wid = lax.axis_index("c") * 16 + lax.axis_index("s")
    base = wid * (N_TOK // 32)

    def tok0(k):
        return base + k * SC_CHUNK

    h_load = {}
    h_sc = {}
    h_load[0] = pltpu.async_copy(fbf_hbm.at[pl.ds(tok0(0), SC_CHUNK)], rows[0], ldsem[0])
    for k in range(_N_DCH):
        b = k % 2
        if k + 1 < _N_DCH:
            if k - 1 >= 0:
                h_sc[k - 1][0].wait()
                h_sc[k - 1][1].wait()
            h_load[k + 1] = pltpu.async_copy(
                fbf_hbm.at[pl.ds(tok0(k + 1), SC_CHUNK)], rows[(k + 1) % 2], ldsem[(k + 1) % 2]
            )
        pltpu.sync_copy(s1_hbm.at[pl.ds(tok0(k), SC_CHUNK)], idx1[b])
        pltpu.sync_copy(s2_hbm.at[pl.ds(tok0(k), SC_CHUNK)], idx2[b])
        h_load[k].wait()
        h_sc[k] = (
            pltpu.async_copy(rows[b], xg_hbm.at[idx1[b]], scsem[b]),
            pltpu.async_copy(rows[b], xg_hbm.at[idx2[b]], scsem[b]),
        )
    for k in (_N_DCH - 2, _N_DCH - 1):
        h_sc[k][0].wait()
        h_sc[k][1].wait()


def _dispatch(fbf, s1, s2):
    mesh = plsc.VectorSubcoreMesh(core_axis_name="c", subcore_axis_name="s")
    k = pl.kernel(
        _dispatch_body,
        out_type=jax.ShapeDtypeStruct((XG_ROWS, D_HALF), jnp.float32),
        mesh=mesh,
        scratch_types=[
            [pltpu.VMEM((SC_CHUNK,), jnp.int32)] * 2,
            [pltpu.VMEM((SC_CHUNK,), jnp.int32)] * 2,
            [pltpu.VMEM((SC_CHUNK, D_HALF), jnp.float32)] * 2,
            [pltpu.SemaphoreType.DMA] * 2,
            [pltpu.SemaphoreType.DMA] * 2,
        ],
    )
    return k(fbf, s1, s2)


def _ffn_body(xg_ref, w1_ref, b1_ref, w2_ref, b2_ref, out_ref, acc):
    f = pl.program_id(1)

    w1b = w1_ref[0].astype(jnp.bfloat16)
    w2b = w2_ref[0].astype(jnp.bfloat16)
    b1v = b1_ref[0]
    b2v = b2_ref[0]
    for c in range(CAP // ROW_BLK):
        u = lax.bitcast_convert_type(xg_ref[pl.ds(c * ROW_BLK, ROW_BLK), :], jnp.uint32)
        xlo = lax.bitcast_convert_type((u & 0xFFFF).astype(jnp.uint16), jnp.bfloat16)
        xhi = lax.bitcast_convert_type((u >> 16).astype(jnp.uint16), jnp.bfloat16)
        h = lax.dot_general(
            xlo, w1b[:D_HALF], (((1,), (0,)), ((), ())), preferred_element_type=jnp.float32
        ) + lax.dot_general(
            xhi, w1b[D_HALF:], (((1,), (0,)), ((), ())), preferred_element_type=jnp.float32
        )
        h = h + b1v
        h = 0.5 * h * (1.0 + lax.erf(h * _RSQRT2))
        y = lax.dot_general(
            h.astype(jnp.bfloat16),
            w2b,
            (((1,), (0,)), ((), ())),
            preferred_element_type=jnp.float32,
        )
        sl = pl.ds(c * ROW_BLK, ROW_BLK)

        @pl.when(f == 0)
        def _():
            acc[sl, :] = y + b2v

        @pl.when(f != 0)
        def _():
            acc[sl, :] += y

        @pl.when(f == N_FF_BLKS - 1)
        def _():
            out_ref[sl, :] = _pack_rows(acc[sl, :])


def _ffn(xg, w1, b1, w2, b2):
    return pl.pallas_call(
        _ffn_body,
        grid=(N_EXP, N_FF_BLKS),
        in_specs=[
            pl.BlockSpec((CAP, D_HALF), lambda e, f: (e, 0)),
            pl.BlockSpec((1, D_MODEL, FF_BLK), lambda e, f: (e, 0, f)),
            pl.BlockSpec((1, 1, FF_BLK), lambda e, f: (e, 0, f)),
            pl.BlockSpec((1, FF_BLK, D_MODEL), lambda e, f: (e, f, 0)),
            pl.BlockSpec((1, 1, D_MODEL), lambda e, f: (e, 0, 0)),
        ],
        out_specs=pl.BlockSpec((CAP, D_HALF), lambda e, f: (e, 0)),
        out_shape=jax.ShapeDtypeStruct((XG_ROWS, D_HALF), jnp.float32),
        scratch_shapes=[pltpu.VMEM((CAP, D_MODEL), jnp.float32)],
    )(xg, w1, b1.reshape(N_EXP, 1, D_FF), w2, b2.reshape(N_EXP, 1, D_MODEL))


CMB_CH = 32
_N_CCH = (N_TOK // 32) // CMB_CH


def _gather_body(out_hbm, s1_hbm, s2_hbm, g1_hbm, g2_hbm,
                 i1, i2, rows1, rows2, gsem, wsem):
    wid = lax.axis_index("c") * 16 + lax.axis_index("s")
    base = wid * (N_TOK // 32)

    def tok0(k):
        return base + k * CMB_CH

    def start_gather(k):
        b = k % 2
        pltpu.sync_copy(s1_hbm.at[pl.ds(tok0(k), CMB_CH)], i1[b])
        pltpu.sync_copy(s2_hbm.at[pl.ds(tok0(k), CMB_CH)], i2[b])
        return (
            pltpu.async_copy(out_hbm.at[i1[b]], rows1[b], gsem[b]),
            pltpu.async_copy(out_hbm.at[i2[b]], rows2[b], gsem[b]),
        )

    h_g = {0: start_gather(0)}
    h_w = {}
    for k in range(_N_CCH):
        b = k % 2
        h_g[k][0].wait()
        h_g[k][1].wait()
        if k - 2 >= 0:
            h_w[k - 2][0].wait()
            h_w[k - 2][1].wait()
        h_w[k] = (
            pltpu.async_copy(rows1[b], g1_hbm.at[pl.ds(tok0(k), CMB_CH)], wsem[b]),
            pltpu.async_copy(rows2[b], g2_hbm.at[pl.ds(tok0(k), CMB_CH)], wsem[b]),
        )
        if k + 1 < _N_CCH:
            h_g[k + 1] = start_gather(k + 1)
    for k in (_N_CCH - 2, _N_CCH - 1):
        h_w[k][0].wait()
        h_w[k][1].wait()


def _gather2(out_ffn, s1, s2):
    mesh = plsc.VectorSubcoreMesh(core_axis_name="c", subcore_axis_name="s")
    k = pl.kernel(
        _gather_body,
        out_type=(
            jax.ShapeDtypeStruct((N_TOK, D_HALF), jnp.float32),
            jax.ShapeDtypeStruct((N_TOK, D_HALF), jnp.float32),
        ),
        mesh=mesh,
        scratch_types=[
            [pltpu.VMEM((CMB_CH,), jnp.int32)] * 2,
            [pltpu.VMEM((CMB_CH,), jnp.int32)] * 2,
            [pltpu.VMEM((CMB_CH, D_HALF), jnp.float32)] * 2,
            [pltpu.VMEM((CMB_CH, D_HALF), jnp.float32)] * 2,
            [pltpu.SemaphoreType.DMA] * 2,
            [pltpu.SemaphoreType.DMA] * 2,
        ],
    )
    return k(out_ffn, s1, s2)


def _combine_body(g1_ref, g2_ref, w1_ref, w2_ref, o_ref):
    w1v = w1_ref[:, :1]
    w2v = w2_ref[:, :1]
    y1 = _unpack_rows(g1_ref[...]).astype(jnp.float32)
    y2 = _unpack_rows(g2_ref[...]).astype(jnp.float32)
    a = jnp.where(w1v > 0, y1, 0.0) * w1v
    b = jnp.where(w2v > 0, y2, 0.0) * w2v
    o_ref[...] = a + b


def _combine(g1, g2, w1x, w2x):
    return pl.pallas_call(
        _combine_body,
        grid=(N_TOK_BLKS,),
        in_specs=[
            pl.BlockSpec((TOK_BLK, D_HALF), lambda i: (i, 0)),
            pl.BlockSpec((TOK_BLK, D_HALF), lambda i: (i, 0)),
            pl.BlockSpec((TOK_BLK, 16), lambda i: (i, 0)),
            pl.BlockSpec((TOK_BLK, 16), lambda i: (i, 0)),
        ],
        out_specs=pl.BlockSpec((TOK_BLK, D_MODEL), lambda i: (i, 0)),
        out_shape=jax.ShapeDtypeStruct((N_TOK, D_MODEL), jnp.float32),
    )(g1, g2, w1x, w2x)


def kernel(hidden_states, Wg, W1, b1, W2, b2):
    bh, th, d = hidden_states.shape
    flat = hidden_states.reshape(bh * th, d)
    s1, s2, w1x, w2x, fpk = _router(flat, Wg)
    s1f = s1.reshape(N_TOK)
    s2f = s2.reshape(N_TOK)
    xg = _dispatch(fpk, s1f, s2f)
    out_ffn = _ffn(xg, W1, b1, W2, b2)
    g1, g2 = _gather2(out_ffn, s1f, s2f)
    final = _combine(g1, g2, w1x, w2x)
    aux_loss = jnp.asarray(0.0, dtype=jnp.float32)
    return final.reshape(bh, th, d), aux_loss


# speedup vs baseline: 1.4598x; 1.0116x over previous
import functools
import math

import jax
import jax.numpy as jnp
from jax import lax
from jax.experimental import pallas as pl
from jax.experimental.pallas import tpu as pltpu
from jax.experimental.pallas import tpu_sc as plsc

N_TOK = 8192
D_MODEL = 1024
D_FF = 4096
N_EXP = 8
TOPK = 2
CAP = 2048
DUMMY = N_EXP * CAP
XG_ROWS = N_EXP * CAP + CAP

TOK_BLK = 1024
N_TOK_BLKS = N_TOK // TOK_BLK
FF_BLK = 1024
N_FF_BLKS = D_FF // FF_BLK
ROW_BLK = 512
D_HALF = D_MODEL // 2


def _pack_rows(x32):
    lo = lax.bitcast_convert_type(x32[:, :D_HALF].astype(jnp.bfloat16), jnp.uint16)
    hi = lax.bitcast_convert_type(x32[:, D_HALF:].astype(jnp.bfloat16), jnp.uint16)
    u = lo.astype(jnp.uint32) | (hi.astype(jnp.uint32) << 16)
    return lax.bitcast_convert_type(u, jnp.float32)


def _unpack_rows(p32):
    u = lax.bitcast_convert_type(p32, jnp.uint32)
    lo = lax.bitcast_convert_type((u & 0xFFFF).astype(jnp.uint16), jnp.bfloat16)
    hi = lax.bitcast_convert_type((u >> 16).astype(jnp.uint16), jnp.bfloat16)
    return jnp.concatenate([lo, hi], axis=1)

SC_CHUNK = 64
_RSQRT2 = 1.0 / math.sqrt(2.0)


def _router_body(x_ref, wg_ref, s1_ref, s2_ref, w1_ref, w2_ref, fp_ref, carry_ref):
    i = pl.program_id(0)

    @pl.when(i == 0)
    def _():
        carry_ref[...] = jnp.zeros((1, N_EXP), jnp.float32)

    x = x_ref[...]
    fp_ref[...] = _pack_rows(x)
    wg = wg_ref[...]
    logits = lax.dot_general(
        x, wg, (((1,), (1,)), ((), ())), preferred_element_type=jnp.float32
    )
    z = logits - jnp.max(logits, axis=1, keepdims=True)
    p = jnp.exp(z)
    gates = p / jnp.sum(p, axis=1, keepdims=True)

    e_iota = lax.broadcasted_iota(jnp.int32, (TOK_BLK, N_EXP), 1)
    v1 = jnp.max(gates, axis=1, keepdims=True)
    i1 = jnp.min(jnp.where(gates == v1, e_iota, N_EXP), axis=1, keepdims=True)
    oh1 = e_iota == i1
    g2 = jnp.where(oh1, -1.0, gates)
    v2 = jnp.max(g2, axis=1, keepdims=True)
    i2 = jnp.min(jnp.where(g2 == v2, e_iota, N_EXP), axis=1, keepdims=True)
    oh2 = e_iota == i2

    mf = (oh1 | oh2).astype(jnp.float32)
    row = lax.broadcasted_iota(jnp.int32, (TOK_BLK, TOK_BLK), 0)
    col = lax.broadcasted_iota(jnp.int32, (TOK_BLK, TOK_BLK), 1)
    trif = (row >= col).astype(jnp.float32)
    incl = lax.dot_general(
        trif, mf, (((1,), (0,)), ((), ())), preferred_element_type=jnp.float32
    )
    pos = carry_ref[...] + incl - 1.0
    keepf = jnp.where(pos < float(CAP), mf, 0.0)
    gsf = e_iota.astype(jnp.float32) * float(CAP) + pos
    gs = jnp.where(keepf > 0, gsf, float(DUMMY))

    slot1 = jnp.sum(jnp.where(oh1, gs, 0.0), axis=1, keepdims=True)
    slot2 = jnp.sum(jnp.where(oh2, gs, 0.0), axis=1, keepdims=True)
    k1 = jnp.sum(jnp.where(oh1, keepf, 0.0), axis=1, keepdims=True)
    k2 = jnp.sum(jnp.where(oh2, keepf, 0.0), axis=1, keepdims=True)

    s1_ref[...] = slot1.astype(jnp.int32)
    s2_ref[...] = slot2.astype(jnp.int32)
    w1_ref[...] = jnp.broadcast_to(v1 * k1, (TOK_BLK, 16))
    w2_ref[...] = jnp.broadcast_to(v2 * k2, (TOK_BLK, 16))
    carry_ref[...] = carry_ref[...] + jnp.sum(mf, axis=0, keepdims=True)


def _router(flat, wg):
    return pl.pallas_call(
        _router_body,
        grid=(N_TOK_BLKS,),
        in_specs=[
            pl.BlockSpec((TOK_BLK, D_MODEL), lambda i: (i, 0)),
            pl.BlockSpec((N_EXP, D_MODEL), lambda i: (0, 0)),
        ],
        out_specs=[
            pl.BlockSpec((TOK_BLK, 1), lambda i: (i, 0)),
            pl.BlockSpec((TOK_BLK, 1), lambda i: (i, 0)),
            pl.BlockSpec((TOK_BLK, 16), lambda i: (i, 0)),
            pl.BlockSpec((TOK_BLK, 16), lambda i: (i, 0)),
            pl.BlockSpec((TOK_BLK, D_HALF), lambda i: (i, 0)),
        ],
        out_shape=[
            jax.ShapeDtypeStruct((N_TOK, 1), jnp.int32),
            jax.ShapeDtypeStruct((N_TOK, 1), jnp.int32),
            jax.ShapeDtypeStruct((N_TOK, 16), jnp.float32),
            jax.ShapeDtypeStruct((N_TOK, 16), jnp.float32),
            jax.ShapeDtypeStruct((N_TOK, D_HALF), jnp.float32),
        ],
        scratch_shapes=[pltpu.VMEM((1, N_EXP), jnp.float32)],
    )(flat, wg)


_N_DCH = (N_TOK // 32) // SC_CHUNK


def _dispatch_body(fbf_hbm, s1_hbm, s2_hbm, xg_hbm, idx1, idx2, rows, ldsem, scsem):
    wid = lax.axis_index("c") * 16 + lax.axis_index("s")
    base = wid * (N_TOK // 32)

    def tok0(k):
        return base + k * SC_CHUNK

    h_load = {}
    h_sc = {}
    h_load[0] = pltpu.async_copy(fbf_hbm.at[pl.ds(tok0(0), SC_CHUNK)], rows[0], ldsem[0])
    for k in range(_N_DCH):
        b = k % 2
        if k + 1 < _N_DCH:
            if k - 1 >= 0:
                h_sc[k - 1][0].wait()
                h_sc[k - 1][1].wait()
            h_load[k + 1] = pltpu.async_copy(
                fbf_hbm.at[pl.ds(tok0(k + 1), SC_CHUNK)], rows[(k + 1) % 2], ldsem[(k + 1) % 2]
            )
        pltpu.sync_copy(s1_hbm.at[pl.ds(tok0(k), SC_CHUNK)], idx1[b])
        pltpu.sync_copy(s2_hbm.at[pl.ds(tok0(k), SC_CHUNK)], idx2[b])
        h_load[k].wait()
        h_sc[k] = (
            pltpu.async_copy(rows[b], xg_hbm.at[idx1[b]], scsem[b]),
            pltpu.async_copy(rows[b], xg_hbm.at[idx2[b]], scsem[b]),
        )
    for k in (_N_DCH - 2, _N_DCH - 1):
        h_sc[k][0].wait()
        h_sc[k][1].wait()


def _dispatch(fbf, s1, s2):
    mesh = plsc.VectorSubcoreMesh(core_axis_name="c", subcore_axis_name="s")
    k = pl.kernel(
        _dispatch_body,
        out_type=jax.ShapeDtypeStruct((XG_ROWS, D_HALF), jnp.float32),
        mesh=mesh,
        scratch_types=[
            [pltpu.VMEM((SC_CHUNK,), jnp.int32)] * 2,
            [pltpu.VMEM((SC_CHUNK,), jnp.int32)] * 2,
            [pltpu.VMEM((SC_CHUNK, D_HALF), jnp.float32)] * 2,
            [pltpu.SemaphoreType.DMA] * 2,
            [pltpu.SemaphoreType.DMA] * 2,
        ],
    )
    return k(fbf, s1, s2)


def _ffn_body(xg_ref, w1_ref, b1_ref, w2_ref, b2_ref, out_ref, acc):
    f = pl.program_id(1)

    w1b = w1_ref[0].astype(jnp.bfloat16)
    w2b = w2_ref[0].astype(jnp.bfloat16)
    b1v = b1_ref[0]
    b2v = b2_ref[0]
    for c in range(CAP // ROW_BLK):
        u = lax.bitcast_convert_type(xg_ref[pl.ds(c * ROW_BLK, ROW_BLK), :], jnp.uint32)
        xlo = lax.bitcast_convert_type((u & 0xFFFF).astype(jnp.uint16), jnp.bfloat16)
        xhi = lax.bitcast_convert_type((u >> 16).astype(jnp.uint16), jnp.bfloat16)
        h = lax.dot_general(
            xlo, w1b[:D_HALF], (((1,), (0,)), ((), ())), preferred_element_type=jnp.float32
        ) + lax.dot_general(
            xhi, w1b[D_HALF:], (((1,), (0,)), ((), ())), preferred_element_type=jnp.float32
        )
        h = h + b1v
        h = 0.5 * h * (1.0 + lax.erf(h * _RSQRT2))
        y = lax.dot_general(
            h.astype(jnp.bfloat16),
            w2b,
            (((1,), (0,)), ((), ())),
            preferred_element_type=jnp.float32,
        )
        sl = pl.ds(c * ROW_BLK, ROW_BLK)

        @pl.when(f == 0)
        def _():
            acc[sl, :] = y + b2v

        @pl.when(f != 0)
        def _():
            acc[sl, :] += y

        @pl.when(f == N_FF_BLKS - 1)
        def _():
            out_ref[sl, :] = _pack_rows(acc[sl, :])


def _ffn(xg, w1, b1, w2, b2):
    return pl.pallas_call(
        _ffn_body,
        grid=(N_EXP, N_FF_BLKS),
        in_specs=[
            pl.BlockSpec((CAP, D_HALF), lambda e, f: (e, 0)),
            pl.BlockSpec((1, D_MODEL, FF_BLK), lambda e, f: (e, 0, f)),
            pl.BlockSpec((1, 1, FF_BLK), lambda e, f: (e, 0, f)),
            pl.BlockSpec((1, FF_BLK, D_MODEL), lambda e, f: (e, f, 0)),
            pl.BlockSpec((1, 1, D_MODEL), lambda e, f: (e, 0, 0)),
        ],
        out_specs=pl.BlockSpec((CAP, D_HALF), lambda e, f: (e, 0)),
        out_shape=jax.ShapeDtypeStruct((XG_ROWS, D_HALF), jnp.float32),
        scratch_shapes=[pltpu.VMEM((CAP, D_MODEL), jnp.float32)],
    )(xg, w1, b1.reshape(N_EXP, 1, D_FF), w2, b2.reshape(N_EXP, 1, D_MODEL))


CMB_CH = 32
_N_CCH = (N_TOK // 32) // CMB_CH


def _gather_body(out_hbm, s1_hbm, s2_hbm, g1_hbm, g2_hbm,
                 i1, i2, rows1, rows2, gsem, wsem):
    wid = lax.axis_index("c") * 16 + lax.axis_index("s")
    base = wid * (N_TOK // 32)

    def tok0(k):
        return base + k * CMB_CH

    def start_gather(k):
        b = k % 2
        pltpu.sync_copy(s1_hbm.at[pl.ds(tok0(k), CMB_CH)], i1[b])
        pltpu.sync_copy(s2_hbm.at[pl.ds(tok0(k), CMB_CH)], i2[b])
        return (
            pltpu.async_copy(out_hbm.at[i1[b]], rows1[b], gsem[b]),
            pltpu.async_copy(out_hbm.at[i2[b]], rows2[b], gsem[b]),
        )

    h_g = {0: start_gather(0)}
    h_w = {}
    for k in range(_N_CCH):
        b = k % 2
        h_g[k][0].wait()
        h_g[k][1].wait()
        if k - 2 >= 0:
            h_w[k - 2][0].wait()
            h_w[k - 2][1].wait()
        h_w[k] = (
            pltpu.async_copy(rows1[b], g1_hbm.at[pl.ds(tok0(k), CMB_CH)], wsem[b]),
            pltpu.async_copy(rows2[b], g2_hbm.at[pl.ds(tok0(k), CMB_CH)], wsem[b]),
        )
        if k + 1 < _N_CCH:
            h_g[k + 1] = start_gather(k + 1)
    for k in (_N_CCH - 2, _N_CCH - 1):
        h_w[k][0].wait()
        h_w[k][1].wait()


def _gather2(out_ffn, s1, s2):
    mesh = plsc.VectorSubcoreMesh(core_axis_name="c", subcore_axis_name="s")
    k = pl.kernel(
        _gather_body,
        out_type=(
            jax.ShapeDtypeStruct((N_TOK, D_HALF), jnp.float32),
            jax.ShapeDtypeStruct((N_TOK, D_HALF), jnp.float32),
        ),
        mesh=mesh,
        scratch_types=[
            [pltpu.VMEM((CMB_CH,), jnp.int32)] * 2,
            [pltpu.VMEM((CMB_CH,), jnp.int32)] * 2,
            [pltpu.VMEM((CMB_CH, D_HALF), jnp.float32)] * 2,
            [pltpu.VMEM((CMB_CH, D_HALF), jnp.float32)] * 2,
            [pltpu.SemaphoreType.DMA] * 2,
            [pltpu.SemaphoreType.DMA] * 2,
        ],
    )
    return k(out_ffn, s1, s2)


def _combine_body(g1_ref, g2_ref, w1_ref, w2_ref, o_ref):
    w1v = w1_ref[:, :1]
    w2v = w2_ref[:, :1]
    y1 = _unpack_rows(g1_ref[...]).astype(jnp.float32)
    y2 = _unpack_rows(g2_ref[...]).astype(jnp.float32)
    a = jnp.where(w1v > 0, y1, 0.0) * w1v
    b = jnp.where(w2v > 0, y2, 0.0) * w2v
    o_ref[...] = a + b


def _combine(g1, g2, w1x, w2x):
    return pl.pallas_call(
        _combine_body,
        grid=(N_TOK_BLKS,),
        in_specs=[
            pl.BlockSpec((TOK_BLK, D_HALF), lambda i: (i, 0)),
            pl.BlockSpec((TOK_BLK, D_HALF), lambda i: (i, 0)),
            pl.BlockSpec((TOK_BLK, 16), lambda i: (i, 0)),
            pl.BlockSpec((TOK_BLK, 16), lambda i: (i, 0)),
        ],
        out_specs=pl.BlockSpec((TOK_BLK, D_MODEL), lambda i: (i, 0)),
        out_shape=jax.ShapeDtypeStruct((N_TOK, D_MODEL), jnp.float32),
    )(g1, g2, w1x, w2x)


def kernel(hidden_states, Wg, W1, b1, W2, b2):
    bh, th, d = hidden_states.shape
    flat = hidden_states.reshape(bh * th, d)
    s1, s2, w1x, w2x, fpk = _router(flat, Wg)
    s1f = s1.reshape(N_TOK)
    s2f = s2.reshape(N_TOK)
    xg = _dispatch(fpk, s1f, s2f)
    out_ffn = _ffn(xg, W1, b1, W2, b2)
    g1, g2 = _gather2(out_ffn, s1f, s2f)
    final = _combine(g1, g2, w1x, w2x)
    aux_loss = jnp.asarray(0.0, dtype=jnp.float32)
    return final.reshape(bh, th, d), aux_loss
